# Initial kernel scaffold; baseline (speedup 1.0000x reference)
import jax
import jax.numpy as jnp
from jax.experimental import pallas as pl


def kernel(x, edge_index, W1, a1s, a1d, b1, W2, a2s, a2d, b2, Wl, bl):
    raise NotImplementedError("write your pallas kernel here")


# speedup vs baseline: 24.4295x; 24.4295x over previous
import functools

import jax
import jax.numpy as jnp
from jax import lax
from jax.experimental import pallas as pl
from jax.experimental.pallas import tpu as pltpu
from jax.experimental.pallas import tpu_sc as plsc

N = 100000
E = 1600000
HEADS = 3
C = 64
HC = HEADS * C

BT = 256
FPT = 13
CO = BT * FPT
NW = 32
NPAD = CO * NW
ROWW = 256
EPW = E // NW
SUB = 10000
NSUB = EPW // SUB
SUBB = 2048
CHK = 64
OFFSL1 = 1040
E2CAP = E + NW * NW * 16 + SUBB
E3CAP = E2CAP + NW * 272 + 128
_GRID = NPAD // 512

_mesh = None


def _get_mesh():
    global _mesh
    if _mesh is None:
        _mesh = plsc.VectorSubcoreMesh(core_axis_name="c", subcore_axis_name="s")
    return _mesh


def _lane():
    return lax.iota(jnp.int32, 16)


def _extract_i32(vec, idx):
    return jnp.sum(jnp.where(_lane() == idx, vec, 0))


def _extract_f32(vec, idx):
    return jnp.sum(jnp.where(_lane() == idx, vec, jnp.float32(0)))


_SC_PARAMS = dict(
    compiler_params=pltpu.CompilerParams(needs_layout_passes=False),
)


def _sc_hist(dst1d):
    @functools.partial(
        pl.kernel,
        mesh=_get_mesh(),
        out_type=jax.ShapeDtypeStruct((NW * 32,), jnp.int32),
        scratch_types=[
            pltpu.VMEM((SUB,), jnp.int32),
            pltpu.VMEM((NW * 16 + 16,), jnp.int32),
            pltpu.VMEM((32,), jnp.int32),
        ],
        name="gat_sc_hist",
        **_SC_PARAMS,
    )
    def k(dst_hbm, counts_hbm, ebuf, hist, cntv):
        wid = lax.axis_index("c") * 16 + lax.axis_index("s")
        for kk in range(NW + 1):
            hist[pl.ds(kk * 16, 16)] = jnp.zeros((16,), jnp.int32)
        ones = jnp.ones((16,), jnp.int32)
        lane = _lane()
        for sub in range(NSUB):
            pltpu.sync_copy(dst_hbm.at[pl.ds(wid * EPW + sub * SUB, SUB)], ebuf)

            def body(i, carry):
                dv = ebuf[pl.ds(i * 16, 16)]
                bv = dv // CO
                idx = bv * 16 + lane
                cur = plsc.load_gather(hist, [idx])
                plsc.store_scatter(hist, [idx], cur + ones)
                return carry

            lax.fori_loop(0, SUB // 16, body, 0)
        cv0 = jnp.zeros((16,), jnp.int32)
        cv1 = jnp.zeros((16,), jnp.int32)
        for b in range(NW):
            sb = jnp.sum(hist[pl.ds(b * 16, 16)])
            if b < 16:
                cv0 = jnp.where(lane == b, sb, cv0)
            else:
                cv1 = jnp.where(lane == (b - 16), sb, cv1)
        cntv[pl.ds(0, 16)] = cv0
        cntv[pl.ds(16, 16)] = cv1
        pltpu.sync_copy(cntv, counts_hbm.at[pl.ds(wid * 32, 32)])

    return k(dst1d)


def _sc_coarse(src1d, dst1d, offs):
    @functools.partial(
        pl.kernel,
        mesh=_get_mesh(),
        out_type=(
            jax.ShapeDtypeStruct((E2CAP,), jnp.int32),
            jax.ShapeDtypeStruct((E2CAP,), jnp.int32),
        ),
        scratch_types=[
            pltpu.VMEM((SUB,), jnp.int32),
            pltpu.VMEM((SUB,), jnp.int32),
            pltpu.VMEM((NW * 48,), jnp.int32),
            pltpu.VMEM((NW * 48,), jnp.int32),
            pltpu.VMEM((OFFSL1,), jnp.int32),
            pltpu.VMEM((16,), jnp.int32),
            pltpu.VMEM((16,), jnp.int32),
        ],
        name="gat_sc_coarse",
        **_SC_PARAMS,
    )
    def k(src_hbm, dst_hbm, offs_hbm, srcb_hbm, dstb_hbm, sbuf, dbuf, sst,
          dst_st, offsv, pads, padd):
        wid = lax.axis_index("c") * 16 + lax.axis_index("s")
        pltpu.sync_copy(offs_hbm, offsv)
        lane = _lane()

        ptrs0 = []
        for b in range(NW):
            fi = b * NW + wid
            base = (fi // 16) * 16
            ov = offsv[pl.ds(base, 16)]
            ptrs0.append(_extract_i32(ov, fi - base))

        carry0 = tuple([jnp.int32(0)] * NW) + tuple(ptrs0)

        def vbody(i, carry):
            cnts = list(carry[:NW])
            ptrs = list(carry[NW:])
            sv = sbuf[pl.ds(i * 16, 16)]
            dv = dbuf[pl.ds(i * 16, 16)]
            bv = dv // CO
            for b in range(NW):
                m = bv == b
                cnt = cnts[b]
                ptr = ptrs[b]
                plsc.store_compressed(sst.at[pl.ds(b * 48 + cnt, 16)], sv, mask=m)
                plsc.store_compressed(dst_st.at[pl.ds(b * 48 + cnt, 16)], dv, mask=m)
                cnt2 = cnt + jnp.sum(m.astype(jnp.int32))
                fire = cnt2 >= 16
                ptr_a = pl.multiple_of(ptr, 16)

                @pl.when(fire)
                def _():
                    pltpu.sync_copy(sst.at[pl.ds(b * 48, 16)],
                                    srcb_hbm.at[pl.ds(ptr_a, 16)])
                    pltpu.sync_copy(dst_st.at[pl.ds(b * 48, 16)],
                                    dstb_hbm.at[pl.ds(ptr_a, 16)])
                    lv = sst[pl.ds(b * 48 + 16, 16)]
                    sst[pl.ds(b * 48, 16)] = lv
                    lv2 = dst_st[pl.ds(b * 48 + 16, 16)]
                    dst_st[pl.ds(b * 48, 16)] = lv2

                fi32 = fire.astype(jnp.int32)
                cnts[b] = cnt2 - 16 * fi32
                ptrs[b] = ptr + 16 * fi32
            return tuple(cnts) + tuple(ptrs)

        carry = carry0
        for sub in range(NSUB):
            pltpu.sync_copy(src_hbm.at[pl.ds(wid * EPW + sub * SUB, SUB)], sbuf)
            pltpu.sync_copy(dst_hbm.at[pl.ds(wid * EPW + sub * SUB, SUB)], dbuf)
            carry = lax.fori_loop(0, SUB // 16, vbody, carry)

        cnts = list(carry[:NW])
        ptrs = list(carry[NW:])
        for b in range(NW):
            fi = b * NW + wid + 1
            base = (fi // 16) * 16
            ov = offsv[pl.ds(base, 16)]
            end = _extract_i32(ov, fi - base)
            cnt = cnts[b]
            ptr = ptrs[b]
            sv = sst[pl.ds(b * 48, 16)]
            dv = dst_st[pl.ds(b * 48, 16)]
            sv = jnp.where(lane < cnt, sv, 0)
            dv = jnp.where(lane < cnt, dv, jnp.int32((b + 1) * CO))
            sst[pl.ds(b * 48, 16)] = sv
            dst_st[pl.ds(b * 48, 16)] = dv
            ptr_a = pl.multiple_of(ptr, 16)

            @pl.when(ptr < end)
            def _():
                pltpu.sync_copy(sst.at[pl.ds(b * 48, 16)],
                                srcb_hbm.at[pl.ds(ptr_a, 16)])
                pltpu.sync_copy(dst_st.at[pl.ds(b * 48, 16)],
                                dstb_hbm.at[pl.ds(ptr_a, 16)])

    return k(src1d, dst1d, offs)


def _sc_fine(srcb, dstb, offs1, foff):
    nfb = FPT + 1

    @functools.partial(
        pl.kernel,
        mesh=_get_mesh(),
        out_type=(
            jax.ShapeDtypeStruct((E3CAP,), jnp.int32),
            jax.ShapeDtypeStruct((E3CAP,), jnp.int32),
            jax.ShapeDtypeStruct((NW * 16,), jnp.int32),
        ),
        scratch_types=[
            pltpu.VMEM((SUBB,), jnp.int32),
            pltpu.VMEM((SUBB,), jnp.int32),
            pltpu.VMEM((nfb * 48,), jnp.int32),
            pltpu.VMEM((nfb * 48,), jnp.int32),
            pltpu.VMEM((OFFSL1,), jnp.int32),
            pltpu.VMEM((48,), jnp.int32),
            pltpu.VMEM((nfb * 16 + 16,), jnp.int32),
            pltpu.VMEM((16,), jnp.int32),
        ],
        name="gat_sc_fine",
        **_SC_PARAMS,
    )
    def k(srcb_hbm, dstb_hbm, offs1_hbm, foff_hbm, src2_hbm, dst2_hbm,
          offs2_hbm, sbuf, dbuf, sst, dst_st, offsv, foffv, hist, o2v):
        t = lax.axis_index("c") * 16 + lax.axis_index("s")
        pltpu.sync_copy(offs1_hbm, offsv)
        pltpu.sync_copy(foff_hbm, foffv)
        lane = _lane()
        clo = t * CO

        fi = t * NW
        ov = offsv[pl.ds(pl.multiple_of((fi // 16) * 16, 16), 16)]
        cst = _extract_i32(ov, fi - (fi // 16) * 16)
        fj = (t + 1) * NW
        ov1 = offsv[pl.ds(pl.multiple_of((fj // 16) * 16, 16), 16)]
        cen = _extract_i32(ov1, fj - (fj // 16) * 16)
        fbase0 = (t // 16) * 16
        fv = foffv[pl.ds(fbase0, 16)]
        fbase = _extract_i32(fv, t - fbase0)

        for kk in range(nfb + 1):
            hist[pl.ds(kk * 16, 16)] = jnp.zeros((16,), jnp.int32)
        ones = jnp.ones((16,), jnp.int32)

        def h_outer(p, carry):
            p_a = pl.multiple_of(p, 16)
            pltpu.sync_copy(dstb_hbm.at[pl.ds(p_a, SUBB)], dbuf)
            rem = cen - p

            def h_body(i, carry2):
                dv = dbuf[pl.ds(i * 16, 16)]
                valid = (lane + i * 16) < rem
                dv = jnp.where(valid, dv, jnp.int32(NPAD))
                bv = jnp.minimum((dv - clo) // BT, FPT)
                idx = bv * 16 + lane
                cur = plsc.load_gather(hist, [idx])
                plsc.store_scatter(hist, [idx], cur + ones)
                return carry2

            lax.fori_loop(0, SUBB // 16, h_body, 0)
            return carry

        pl.loop(cst, cen, step=SUBB, init_carry=0)(h_outer)

        starts = []
        sacc = fbase
        for f in range(FPT):
            starts.append(sacc)
            cf = jnp.sum(hist[pl.ds(f * 16, 16)])
            sacc = sacc + ((cf + 15) // 16) * 16

        o2 = jnp.zeros((16,), jnp.int32)
        for f in range(FPT):
            o2 = jnp.where(lane == f, starts[f], o2)
        o2 = jnp.where(lane == FPT, sacc, o2)
        o2v[...] = o2
        pltpu.sync_copy(o2v, offs2_hbm.at[pl.ds(pl.multiple_of(t * 16, 16), 16)])

        def vbody(i, carry):
            rem, cnts_ptrs = carry
            cnts = list(cnts_ptrs[:FPT])
            ptrs = list(cnts_ptrs[FPT:])
            sv = sbuf[pl.ds(i * 16, 16)]
            dv = dbuf[pl.ds(i * 16, 16)]
            valid = (lane + i * 16) < rem
            sv = jnp.where(valid, sv, 0)
            dv = jnp.where(valid, dv, jnp.int32(clo + CO))
            bv = jnp.minimum((dv - clo) // BT, FPT)
            for f in range(FPT):
                m = bv == f
                cnt = cnts[f]
                ptr = ptrs[f]
                plsc.store_compressed(sst.at[pl.ds(f * 48 + cnt, 16)], sv, mask=m)
                plsc.store_compressed(dst_st.at[pl.ds(f * 48 + cnt, 16)], dv, mask=m)
                cnt2 = cnt + jnp.sum(m.astype(jnp.int32))
                fire = cnt2 >= 16
                ptr_a = pl.multiple_of(ptr, 16)

                @pl.when(fire)
                def _():
                    pltpu.sync_copy(sst.at[pl.ds(f * 48, 16)],
                                    src2_hbm.at[pl.ds(ptr_a, 16)])
                    pltpu.sync_copy(dst_st.at[pl.ds(f * 48, 16)],
                                    dst2_hbm.at[pl.ds(ptr_a, 16)])
                    lv = sst[pl.ds(f * 48 + 16, 16)]
                    sst[pl.ds(f * 48, 16)] = lv
                    lv2 = dst_st[pl.ds(f * 48 + 16, 16)]
                    dst_st[pl.ds(f * 48, 16)] = lv2

                fi32 = fire.astype(jnp.int32)
                cnts[f] = cnt2 - 16 * fi32
                ptrs[f] = ptr + 16 * fi32
            return rem, tuple(cnts) + tuple(ptrs)

        def outer(p, carry):
            p_a = pl.multiple_of(p, 16)
            pltpu.sync_copy(srcb_hbm.at[pl.ds(p_a, SUBB)], sbuf)
            pltpu.sync_copy(dstb_hbm.at[pl.ds(p_a, SUBB)], dbuf)
            rem, cnts_ptrs = carry
            _, cnts_ptrs = lax.fori_loop(
                0, SUBB // 16, vbody, (cen - p, cnts_ptrs))
            return rem, cnts_ptrs

        carry0 = (jnp.int32(0),
                  tuple([jnp.int32(0)] * FPT) + tuple(starts))
        carry = pl.loop(cst, cen, step=SUBB, init_carry=carry0)(outer)
        if carry is None:
            carry = carry0
        _, cnts_ptrs = carry
        cnts = list(cnts_ptrs[:FPT])
        ptrs = list(cnts_ptrs[FPT:])

        for f in range(FPT):
            cnt = cnts[f]
            ptr = ptrs[f]
            sv = sst[pl.ds(f * 48, 16)]
            dv = dst_st[pl.ds(f * 48, 16)]
            sv = jnp.where(lane < cnt, sv, 0)
            dv = jnp.where(lane < cnt, dv, clo + (f + 1) * BT)
            sst[pl.ds(f * 48, 16)] = sv
            dst_st[pl.ds(f * 48, 16)] = dv
            ptr_a = pl.multiple_of(ptr, 16)

            @pl.when(cnt > 0)
            def _():
                pltpu.sync_copy(sst.at[pl.ds(f * 48, 16)],
                                src2_hbm.at[pl.ds(ptr_a, 16)])
                pltpu.sync_copy(dst_st.at[pl.ds(f * 48, 16)],
                                dst2_hbm.at[pl.ds(ptr_a, 16)])

    return k(srcb, dstb, offs1, foff)


def _sc_edge_pass(g, d, src2, dst2, offs2, m16, outinit):
    @functools.partial(
        pl.kernel,
        mesh=_get_mesh(),
        out_type=jax.ShapeDtypeStruct((NPAD, ROWW), jnp.float32),
        scratch_types=[
            pltpu.VMEM((CHK,), jnp.int32),
            pltpu.VMEM((CHK,), jnp.int32),
            pltpu.VMEM((CHK, ROWW), jnp.float32),
            pltpu.VMEM((BT + 16, 4), jnp.float32),
            pltpu.VMEM((16,), jnp.int32),
            pltpu.VMEM((16,), jnp.float32),
            pltpu.VMEM((CHK * 8,), jnp.float32),
            pltpu.VMEM((BT + 16, ROWW), jnp.float32),
            pltpu.SemaphoreType.DMA,
        ],
        name="gat_sc_edge",
        **_SC_PARAMS,
    )
    def k(g_hbm, d_hbm, src2_hbm, dst2_hbm, offs2_hbm, m_hbm, oi_hbm, out_hbm,
          sidx, didx, rows, dblk, o2v, mv, wbuf, acc, sem):
        t = lax.axis_index("c") * 16 + lax.axis_index("s")
        pltpu.sync_copy(offs2_hbm.at[pl.ds(pl.multiple_of(t * 16, 16), 16)], o2v)
        pltpu.sync_copy(m_hbm, mv)
        ovec = o2v[...]
        mvec = mv[...]
        m0 = _extract_f32(mvec, 0)
        m1 = _extract_f32(mvec, 1)
        m2 = _extract_f32(mvec, 2)
        lane = _lane()
        wsel0 = lane == 4
        wsel1 = lane == 5
        wsel2 = lane == 6

        for f in range(FPT):
            lo = pl.multiple_of(t * CO + f * BT, BT)
            pltpu.sync_copy(oi_hbm.at[pl.ds(lo, BT)], acc.at[pl.ds(0, BT)])
            pltpu.sync_copy(d_hbm.at[pl.ds(lo, BT + 16)], dblk)
            st = _extract_i32(ovec, f)
            en = _extract_i32(ovec, f + 1)

            @pl.loop(st, en, step=CHK)
            def _(p):
                p_a = pl.multiple_of(p, 16)
                pltpu.sync_copy(src2_hbm.at[pl.ds(p_a, CHK)], sidx)
                pltpu.sync_copy(dst2_hbm.at[pl.ds(p_a, CHK)], didx)
                rem = en - p
                for j in range(CHK // 16):
                    dv = didx[pl.ds(j * 16, 16)]
                    sv = sidx[pl.ds(j * 16, 16)]
                    valid = (lane + j * 16) < rem
                    dv = jnp.where(valid, dv - lo, jnp.int32(BT))
                    sv = jnp.where(valid, sv, 0)
                    didx[pl.ds(j * 16, 16)] = dv
                    sidx[pl.ds(j * 16, 16)] = sv
                pltpu.async_copy(g_hbm.at[sidx], rows, sem).wait()
                for j in range(CHK // 16):
                    dv = didx[pl.ds(j * 16, 16)]
                    erow = lane + (j * 16)
                    c192 = jnp.full((16,), 192, jnp.int32)
                    asrc0 = plsc.load_gather(rows, [erow, c192])
                    asrc1 = plsc.load_gather(rows, [erow, c192 + 1])
                    asrc2 = plsc.load_gather(rows, [erow, c192 + 2])
                    czero = jnp.zeros((16,), jnp.int32)
                    adst0 = plsc.load_gather(dblk, [dv, czero])
                    adst1 = plsc.load_gather(dblk, [dv, czero + 1])
                    adst2 = plsc.load_gather(dblk, [dv, czero + 2])

                    def wcalc(a, ad, m):
                        e = a + ad
                        e = jnp.where(e > 0, e, 0.2 * e)
                        return jnp.exp(e - m)

                    w0 = wcalc(asrc0, adst0, m0)
                    w1 = wcalc(asrc1, adst1, m1)
                    w2 = wcalc(asrc2, adst2, m2)
                    e8 = erow * 8
                    plsc.store_scatter(wbuf, [e8], w0)
                    plsc.store_scatter(wbuf, [e8 + 1], w1)
                    plsc.store_scatter(wbuf, [e8 + 2], w2)

                @pl.loop(0, CHK)
                def _(e):
                    w0s = plsc.load_gather(wbuf, [jnp.full((16,), e * 8, jnp.int32)])
                    w1s = plsc.load_gather(wbuf, [jnp.full((16,), e * 8 + 1, jnp.int32)])
                    w2s = plsc.load_gather(wbuf, [jnp.full((16,), e * 8 + 2, jnp.int32)])
                    dls = plsc.load_gather(didx, [jnp.full((16,), e, jnp.int32)])
                    ef = jnp.full((16,), e, jnp.int32)
                    ws = (w0s, w1s, w2s)
                    for kk in range(12):
                        col = kk * 16 + lane
                        x = plsc.load_gather(rows, [ef, col])
                        plsc.addupdate_scatter(acc, [dls, col], x * ws[kk // 4])
                    col = 192 + lane
                    x = plsc.load_gather(rows, [ef, col])
                    wsp = jnp.where(wsel0, w0s, jnp.float32(0))
                    wsp = jnp.where(wsel1, w1s, wsp)
                    wsp = jnp.where(wsel2, w2s, wsp)
                    plsc.addupdate_scatter(acc, [dls, col], x * wsp)

            pltpu.sync_copy(acc.at[pl.ds(0, BT)], out_hbm.at[pl.ds(lo, BT)])

    return k(g, d, src2, dst2, offs2, m16, outinit)


def _tc_max1(xp, wa):
    def body(x_ref, wa_ref, m_ref):
        a = jnp.dot(x_ref[...], wa_ref[...], preferred_element_type=jnp.float32)
        bm = jnp.max(a, axis=0, keepdims=True)

        @pl.when(pl.program_id(0) == 0)
        def _():
            m_ref[...] = jnp.full((8, 128), -jnp.inf, jnp.float32)

        m_ref[...] = jnp.maximum(m_ref[...], bm)

    return pl.pallas_call(
        body,
        grid=(_GRID,),
        in_specs=[
            pl.BlockSpec((512, 128), lambda i: (i, 0)),
            pl.BlockSpec((128, 128), lambda i: (0, 0)),
        ],
        out_specs=pl.BlockSpec((8, 128), lambda i: (0, 0)),
        out_shape=jax.ShapeDtypeStruct((8, 128), jnp.float32),
    )(xp, wa)


def _finalize_block(o, brow):
    num = o[:, 0:HC]
    den = o[:, 196:199]
    den3 = lax.broadcast_in_dim(den, (o.shape[0], HEADS, C), (0, 1))
    y = num.reshape(o.shape[0], HEADS, C) / (den3 + 1e-20)
    return y.reshape(o.shape[0], HC) + brow[0:1, :]


def _tc_max2(out1, b1row, wa2):
    def body(o_ref, b_ref, wa_ref, m_ref):
        y = _finalize_block(o_ref[...], b_ref[...])
        a = jnp.dot(y, wa_ref[...], preferred_element_type=jnp.float32)
        bm = jnp.max(a, axis=0, keepdims=True)

        @pl.when(pl.program_id(0) == 0)
        def _():
            m_ref[...] = jnp.full((8, 128), -jnp.inf, jnp.float32)

        m_ref[...] = jnp.maximum(m_ref[...], bm)

    return pl.pallas_call(
        body,
        grid=(_GRID,),
        in_specs=[
            pl.BlockSpec((512, ROWW), lambda i: (i, 0)),
            pl.BlockSpec((8, HC), lambda i: (0, 0)),
            pl.BlockSpec((HC, 128), lambda i: (0, 0)),
        ],
        out_specs=pl.BlockSpec((8, 128), lambda i: (0, 0)),
        out_shape=jax.ShapeDtypeStruct((8, 128), jnp.float32),
    )(out1, b1row, wa2)


def _assemble(h, asrc, adst, w, g_ref, d_ref, oi_ref):
    rows = h.shape[0]
    z1 = jnp.zeros((rows, 1), jnp.float32)
    ztail = jnp.zeros((rows, ROWW - HC - 7), jnp.float32)
    z4 = jnp.zeros((rows, 4), jnp.float32)
    on3 = jnp.ones((rows, 3), jnp.float32)
    g_ref[:, 0:HC] = h
    g_ref[:, HC:ROWW] = jnp.concatenate([asrc, z1, on3, ztail], axis=1)
    d_ref[...] = jnp.concatenate([adst, z1], axis=1)
    w3 = lax.broadcast_in_dim(w, (rows, HEADS, C), (0, 1))
    hw = (h.reshape(rows, HEADS, C) * w3).reshape(rows, HC)
    oi_ref[:, 0:HC] = hw
    oi_ref[:, HC:ROWW] = jnp.concatenate([z4, w, ztail], axis=1)


def _selfw(a, m_ref):
    asrc = a[:, 0:3]
    adst = a[:, 3:6]
    e = asrc + adst
    e = jnp.where(e > 0, e, 0.2 * e)
    m3 = m_ref[0:1, 0:3] + m_ref[0:1, 3:6]
    w = jnp.exp(e - m3)
    return asrc, adst, w


def _tc_prep1(xp, w1p, acat, m8):
    def body(x_ref, w_ref, a_ref, m_ref, g_ref, d_ref, oi_ref):
        x = x_ref[...]
        h = jnp.dot(x, w_ref[...], preferred_element_type=jnp.float32)
        a = jnp.dot(h, a_ref[...], preferred_element_type=jnp.float32)
        asrc, adst, w = _selfw(a, m_ref)
        _assemble(h, asrc, adst, w, g_ref, d_ref, oi_ref)

    return pl.pallas_call(
        body,
        grid=(_GRID,),
        in_specs=[
            pl.BlockSpec((512, 128), lambda i: (i, 0)),
            pl.BlockSpec((128, HC), lambda i: (0, 0)),
            pl.BlockSpec((HC, 128), lambda i: (0, 0)),
            pl.BlockSpec((8, 128), lambda i: (0, 0)),
        ],
        out_specs=[
            pl.BlockSpec((512, ROWW), lambda i: (i, 0)),
            pl.BlockSpec((512, 4), lambda i: (i, 0)),
            pl.BlockSpec((512, ROWW), lambda i: (i, 0)),
        ],
        out_shape=[
            jax.ShapeDtypeStruct((NPAD, ROWW), jnp.float32),
            jax.ShapeDtypeStruct((NPAD, 4), jnp.float32),
            jax.ShapeDtypeStruct((NPAD, ROWW), jnp.float32),
        ],
    )(xp, w1p, acat, m8)


def _tc_prep2(out1, b1row, w2, acat, m8):
    def body(o_ref, b_ref, w_ref, a_ref, m_ref, g_ref, d_ref, oi_ref):
        y = _finalize_block(o_ref[...], b_ref[...])
        h = jnp.dot(y, w_ref[...], preferred_element_type=jnp.float32)
        a = jnp.dot(h, a_ref[...], preferred_element_type=jnp.float32)
        asrc, adst, w = _selfw(a, m_ref)
        _assemble(h, asrc, adst, w, g_ref, d_ref, oi_ref)

    return pl.pallas_call(
        body,
        grid=(_GRID,),
        in_specs=[
            pl.BlockSpec((512, ROWW), lambda i: (i, 0)),
            pl.BlockSpec((8, HC), lambda i: (0, 0)),
            pl.BlockSpec((HC, HC), lambda i: (0, 0)),
            pl.BlockSpec((HC, 128), lambda i: (0, 0)),
            pl.BlockSpec((8, 128), lambda i: (0, 0)),
        ],
        out_specs=[
            pl.BlockSpec((512, ROWW), lambda i: (i, 0)),
            pl.BlockSpec((512, 4), lambda i: (i, 0)),
            pl.BlockSpec((512, ROWW), lambda i: (i, 0)),
        ],
        out_shape=[
            jax.ShapeDtypeStruct((NPAD, ROWW), jnp.float32),
            jax.ShapeDtypeStruct((NPAD, 4), jnp.float32),
            jax.ShapeDtypeStruct((NPAD, ROWW), jnp.float32),
        ],
    )(out1, b1row, w2, acat, m8)


def _tc_head(out2, b2row, wl8, bl8):
    def body(o_ref, b_ref, wl_ref, bl_ref, y_ref):
        y = _finalize_block(o_ref[...], b_ref[...])
        y_ref[...] = (
            jnp.dot(y, wl_ref[...], preferred_element_type=jnp.float32)
            + bl_ref[0:1, :]
        )

    return pl.pallas_call(
        body,
        grid=(_GRID,),
        in_specs=[
            pl.BlockSpec((512, ROWW), lambda i: (i, 0)),
            pl.BlockSpec((8, HC), lambda i: (0, 0)),
            pl.BlockSpec((HC, 8), lambda i: (0, 0)),
            pl.BlockSpec((8, 8), lambda i: (0, 0)),
        ],
        out_specs=pl.BlockSpec((512, 8), lambda i: (i, 0)),
        out_shape=jax.ShapeDtypeStruct((NPAD, 8), jnp.float32),
    )(out2, b2row, wl8, bl8)


def _acat(a_s, a_d):
    blocks = []
    for h in range(HEADS):
        col_s = jnp.zeros((HEADS, C, 1), jnp.float32).at[h, :, 0].set(a_s[0, h])
        blocks.append(col_s.reshape(HC, 1))
    for h in range(HEADS):
        col_d = jnp.zeros((HEADS, C, 1), jnp.float32).at[h, :, 0].set(a_d[0, h])
        blocks.append(col_d.reshape(HC, 1))
    cat = jnp.concatenate(blocks, axis=1)
    return jnp.pad(cat, ((0, 0), (0, 122)))


def _m16(m8):
    m3 = m8[0, 0:3] + m8[0, 3:6]
    return jnp.pad(m3, (0, 13))


def kernel(x, edge_index, W1, a1s, a1d, b1, W2, a2s, a2d, b2, Wl, bl):
    xp = jnp.pad(x, ((0, NPAD - N), (0, 128 - x.shape[1])))
    w1p = jnp.pad(W1, ((0, 128 - W1.shape[0]), (0, 0)))
    acat1 = _acat(a1s, a1d)
    acat2 = _acat(a2s, a2d)
    wa1 = w1p @ acat1
    wa2 = W2 @ acat2
    b1row = jnp.broadcast_to(b1[None, :], (8, HC))
    b2row = jnp.broadcast_to(b2[None, :], (8, HC))
    wl8 = jnp.pad(Wl, ((0, 0), (0, 7)))
    bl8 = jnp.broadcast_to(jnp.pad(bl, (0, 7))[None, :], (8, 8))

    src1d = edge_index[0]
    dst1d = edge_index[1]
    counts = _sc_hist(dst1d).reshape(NW, 32)
    cb = counts.T.reshape(-1).astype(jnp.int32)
    r16 = ((cb + 15) // 16) * 16
    offs1 = jnp.concatenate([jnp.zeros((1,), jnp.int32), jnp.cumsum(r16)])
    offs1 = jnp.pad(offs1, (0, OFFSL1 - NW * NW - 1)).astype(jnp.int32)
    srcb, dstb = _sc_coarse(src1d, dst1d, offs1)
    foff = offs1[jnp.arange(NW) * NW] + jnp.arange(NW, dtype=jnp.int32) * 272
    foff = jnp.pad(foff, (0, 16)).astype(jnp.int32)
    src2, dst2, offs2 = _sc_fine(srcb, dstb, offs1, foff)

    m8_1 = _tc_max1(xp, wa1)
    g1, d1, oi1 = _tc_prep1(xp, w1p, acat1, m8_1)
    d1p = jnp.pad(d1, ((0, 16), (0, 0)))
    out1 = _sc_edge_pass(g1, d1p, src2, dst2, offs2, _m16(m8_1), oi1)

    m8_2 = _tc_max2(out1, b1row, wa2)
    g2, d2, oi2 = _tc_prep2(out1, b1row, W2, acat2, m8_2)
    d2p = jnp.pad(d2, ((0, 16), (0, 0)))
    out2 = _sc_edge_pass(g2, d2p, src2, dst2, offs2, _m16(m8_2), oi2)

    y = _tc_head(out2, b2row, wl8, bl8)
    return y[:N, 0:1]


# speedup vs baseline: 25.3989x; 1.0397x over previous
import functools

import jax
import jax.numpy as jnp
from jax import lax
from jax.experimental import pallas as pl
from jax.experimental.pallas import tpu as pltpu
from jax.experimental.pallas import tpu_sc as plsc

N = 100000
E = 1600000
HEADS = 3
C = 64
HC = HEADS * C

BT = 256
FPT = 13
CO = BT * FPT
NW = 32
NPAD = CO * NW
ROWW = 256
EPW = E // NW
SUB = 10000
NSUB = EPW // SUB
SUBB = 2048
CHK = 64
OFFSL1 = 1040
E2CAP = E + NW * NW * 16 + SUBB
E3CAP = E2CAP + NW * 272 + 128
_GRID = NPAD // 512

_mesh = None


def _get_mesh():
    global _mesh
    if _mesh is None:
        _mesh = plsc.VectorSubcoreMesh(core_axis_name="c", subcore_axis_name="s")
    return _mesh


def _lane():
    return lax.iota(jnp.int32, 16)


def _extract_i32(vec, idx):
    return jnp.sum(jnp.where(_lane() == idx, vec, 0))


def _extract_f32(vec, idx):
    return jnp.sum(jnp.where(_lane() == idx, vec, jnp.float32(0)))


_SC_PARAMS = dict(
    compiler_params=pltpu.CompilerParams(needs_layout_passes=False),
)


def _sc_hist(dst1d):
    @functools.partial(
        pl.kernel,
        mesh=_get_mesh(),
        out_type=jax.ShapeDtypeStruct((NW * 32,), jnp.int32),
        scratch_types=[
            pltpu.VMEM((SUB,), jnp.int32),
            pltpu.VMEM((NW * 16 + 16,), jnp.int32),
            pltpu.VMEM((32,), jnp.int32),
        ],
        name="gat_sc_hist",
        **_SC_PARAMS,
    )
    def k(dst_hbm, counts_hbm, ebuf, hist, cntv):
        wid = lax.axis_index("c") * 16 + lax.axis_index("s")
        for kk in range(NW + 1):
            hist[pl.ds(kk * 16, 16)] = jnp.zeros((16,), jnp.int32)
        ones = jnp.ones((16,), jnp.int32)
        lane = _lane()
        for sub in range(NSUB):
            pltpu.sync_copy(dst_hbm.at[pl.ds(wid * EPW + sub * SUB, SUB)], ebuf)

            def body(i, carry):
                dv = ebuf[pl.ds(i * 16, 16)]
                bv = dv // CO
                idx = bv * 16 + lane
                cur = plsc.load_gather(hist, [idx])
                plsc.store_scatter(hist, [idx], cur + ones)
                return carry

            lax.fori_loop(0, SUB // 16, body, 0)
        cv0 = jnp.zeros((16,), jnp.int32)
        cv1 = jnp.zeros((16,), jnp.int32)
        for b in range(NW):
            sb = jnp.sum(hist[pl.ds(b * 16, 16)])
            if b < 16:
                cv0 = jnp.where(lane == b, sb, cv0)
            else:
                cv1 = jnp.where(lane == (b - 16), sb, cv1)
        cntv[pl.ds(0, 16)] = cv0
        cntv[pl.ds(16, 16)] = cv1
        pltpu.sync_copy(cntv, counts_hbm.at[pl.ds(wid * 32, 32)])

    return k(dst1d)


def _sc_coarse(src1d, dst1d, offs):
    @functools.partial(
        pl.kernel,
        mesh=_get_mesh(),
        out_type=(
            jax.ShapeDtypeStruct((E2CAP,), jnp.int32),
            jax.ShapeDtypeStruct((E2CAP,), jnp.int32),
        ),
        scratch_types=[
            pltpu.VMEM((SUB,), jnp.int32),
            pltpu.VMEM((SUB,), jnp.int32),
            pltpu.VMEM((NW * 48,), jnp.int32),
            pltpu.VMEM((NW * 48,), jnp.int32),
            pltpu.VMEM((OFFSL1,), jnp.int32),
            pltpu.VMEM((16,), jnp.int32),
            pltpu.VMEM((16,), jnp.int32),
        ],
        name="gat_sc_coarse",
        **_SC_PARAMS,
    )
    def k(src_hbm, dst_hbm, offs_hbm, srcb_hbm, dstb_hbm, sbuf, dbuf, sst,
          dst_st, offsv, pads, padd):
        wid = lax.axis_index("c") * 16 + lax.axis_index("s")
        pltpu.sync_copy(offs_hbm, offsv)
        lane = _lane()

        ptrs0 = []
        for b in range(NW):
            fi = b * NW + wid
            base = (fi // 16) * 16
            ov = offsv[pl.ds(base, 16)]
            ptrs0.append(_extract_i32(ov, fi - base))

        carry0 = tuple([jnp.int32(0)] * NW) + tuple(ptrs0)

        def vbody(i, carry):
            cnts = list(carry[:NW])
            ptrs = list(carry[NW:])
            sv = sbuf[pl.ds(i * 16, 16)]
            dv = dbuf[pl.ds(i * 16, 16)]
            bv = dv // CO
            for b in range(NW):
                m = bv == b
                cnt = cnts[b]
                ptr = ptrs[b]
                plsc.store_compressed(sst.at[pl.ds(b * 48 + cnt, 16)], sv, mask=m)
                plsc.store_compressed(dst_st.at[pl.ds(b * 48 + cnt, 16)], dv, mask=m)
                cnt2 = cnt + jnp.sum(m.astype(jnp.int32))
                fire = cnt2 >= 16
                ptr_a = pl.multiple_of(ptr, 16)

                @pl.when(fire)
                def _():
                    pltpu.sync_copy(sst.at[pl.ds(b * 48, 16)],
                                    srcb_hbm.at[pl.ds(ptr_a, 16)])
                    pltpu.sync_copy(dst_st.at[pl.ds(b * 48, 16)],
                                    dstb_hbm.at[pl.ds(ptr_a, 16)])
                    lv = sst[pl.ds(b * 48 + 16, 16)]
                    sst[pl.ds(b * 48, 16)] = lv
                    lv2 = dst_st[pl.ds(b * 48 + 16, 16)]
                    dst_st[pl.ds(b * 48, 16)] = lv2

                fi32 = fire.astype(jnp.int32)
                cnts[b] = cnt2 - 16 * fi32
                ptrs[b] = ptr + 16 * fi32
            return tuple(cnts) + tuple(ptrs)

        carry = carry0
        for sub in range(NSUB):
            pltpu.sync_copy(src_hbm.at[pl.ds(wid * EPW + sub * SUB, SUB)], sbuf)
            pltpu.sync_copy(dst_hbm.at[pl.ds(wid * EPW + sub * SUB, SUB)], dbuf)
            carry = lax.fori_loop(0, SUB // 16, vbody, carry)

        cnts = list(carry[:NW])
        ptrs = list(carry[NW:])
        for b in range(NW):
            fi = b * NW + wid + 1
            base = (fi // 16) * 16
            ov = offsv[pl.ds(base, 16)]
            end = _extract_i32(ov, fi - base)
            cnt = cnts[b]
            ptr = ptrs[b]
            sv = sst[pl.ds(b * 48, 16)]
            dv = dst_st[pl.ds(b * 48, 16)]
            sv = jnp.where(lane < cnt, sv, 0)
            dv = jnp.where(lane < cnt, dv, jnp.int32((b + 1) * CO))
            sst[pl.ds(b * 48, 16)] = sv
            dst_st[pl.ds(b * 48, 16)] = dv
            ptr_a = pl.multiple_of(ptr, 16)

            @pl.when(ptr < end)
            def _():
                pltpu.sync_copy(sst.at[pl.ds(b * 48, 16)],
                                srcb_hbm.at[pl.ds(ptr_a, 16)])
                pltpu.sync_copy(dst_st.at[pl.ds(b * 48, 16)],
                                dstb_hbm.at[pl.ds(ptr_a, 16)])

    return k(src1d, dst1d, offs)


def _sc_fine(srcb, dstb, offs1, foff):
    nfb = FPT + 1

    @functools.partial(
        pl.kernel,
        mesh=_get_mesh(),
        out_type=(
            jax.ShapeDtypeStruct((E3CAP,), jnp.int32),
            jax.ShapeDtypeStruct((E3CAP,), jnp.int32),
            jax.ShapeDtypeStruct((NW * 16,), jnp.int32),
        ),
        scratch_types=[
            pltpu.VMEM((SUBB,), jnp.int32),
            pltpu.VMEM((SUBB,), jnp.int32),
            pltpu.VMEM((nfb * 48,), jnp.int32),
            pltpu.VMEM((nfb * 48,), jnp.int32),
            pltpu.VMEM((OFFSL1,), jnp.int32),
            pltpu.VMEM((48,), jnp.int32),
            pltpu.VMEM((nfb * 16 + 16,), jnp.int32),
            pltpu.VMEM((16,), jnp.int32),
        ],
        name="gat_sc_fine",
        **_SC_PARAMS,
    )
    def k(srcb_hbm, dstb_hbm, offs1_hbm, foff_hbm, src2_hbm, dst2_hbm,
          offs2_hbm, sbuf, dbuf, sst, dst_st, offsv, foffv, hist, o2v):
        t = lax.axis_index("c") * 16 + lax.axis_index("s")
        pltpu.sync_copy(offs1_hbm, offsv)
        pltpu.sync_copy(foff_hbm, foffv)
        lane = _lane()
        clo = t * CO

        fi = t * NW
        ov = offsv[pl.ds(pl.multiple_of((fi // 16) * 16, 16), 16)]
        cst = _extract_i32(ov, fi - (fi // 16) * 16)
        fj = (t + 1) * NW
        ov1 = offsv[pl.ds(pl.multiple_of((fj // 16) * 16, 16), 16)]
        cen = _extract_i32(ov1, fj - (fj // 16) * 16)
        fbase0 = (t // 16) * 16
        fv = foffv[pl.ds(fbase0, 16)]
        fbase = _extract_i32(fv, t - fbase0)

        for kk in range(nfb + 1):
            hist[pl.ds(kk * 16, 16)] = jnp.zeros((16,), jnp.int32)
        ones = jnp.ones((16,), jnp.int32)

        def h_outer(p, carry):
            p_a = pl.multiple_of(p, 16)
            pltpu.sync_copy(dstb_hbm.at[pl.ds(p_a, SUBB)], dbuf)
            rem = cen - p

            def h_body(i, carry2):
                dv = dbuf[pl.ds(i * 16, 16)]
                valid = (lane + i * 16) < rem
                dv = jnp.where(valid, dv, jnp.int32(NPAD))
                bv = jnp.minimum((dv - clo) // BT, FPT)
                idx = bv * 16 + lane
                cur = plsc.load_gather(hist, [idx])
                plsc.store_scatter(hist, [idx], cur + ones)
                return carry2

            lax.fori_loop(0, SUBB // 16, h_body, 0)
            return carry

        pl.loop(cst, cen, step=SUBB, init_carry=0)(h_outer)

        starts = []
        sacc = fbase
        for f in range(FPT):
            starts.append(sacc)
            cf = jnp.sum(hist[pl.ds(f * 16, 16)])
            sacc = sacc + ((cf + 15) // 16) * 16

        o2 = jnp.zeros((16,), jnp.int32)
        for f in range(FPT):
            o2 = jnp.where(lane == f, starts[f], o2)
        o2 = jnp.where(lane == FPT, sacc, o2)
        o2v[...] = o2
        pltpu.sync_copy(o2v, offs2_hbm.at[pl.ds(pl.multiple_of(t * 16, 16), 16)])

        def vbody(i, carry):
            rem, cnts_ptrs = carry
            cnts = list(cnts_ptrs[:FPT])
            ptrs = list(cnts_ptrs[FPT:])
            sv = sbuf[pl.ds(i * 16, 16)]
            dv = dbuf[pl.ds(i * 16, 16)]
            valid = (lane + i * 16) < rem
            sv = jnp.where(valid, sv, 0)
            dv = jnp.where(valid, dv, jnp.int32(clo + CO))
            bv = jnp.minimum((dv - clo) // BT, FPT)
            for f in range(FPT):
                m = bv == f
                cnt = cnts[f]
                ptr = ptrs[f]
                plsc.store_compressed(sst.at[pl.ds(f * 48 + cnt, 16)], sv, mask=m)
                plsc.store_compressed(dst_st.at[pl.ds(f * 48 + cnt, 16)], dv, mask=m)
                cnt2 = cnt + jnp.sum(m.astype(jnp.int32))
                fire = cnt2 >= 16
                ptr_a = pl.multiple_of(ptr, 16)

                @pl.when(fire)
                def _():
                    pltpu.sync_copy(sst.at[pl.ds(f * 48, 16)],
                                    src2_hbm.at[pl.ds(ptr_a, 16)])
                    pltpu.sync_copy(dst_st.at[pl.ds(f * 48, 16)],
                                    dst2_hbm.at[pl.ds(ptr_a, 16)])
                    lv = sst[pl.ds(f * 48 + 16, 16)]
                    sst[pl.ds(f * 48, 16)] = lv
                    lv2 = dst_st[pl.ds(f * 48 + 16, 16)]
                    dst_st[pl.ds(f * 48, 16)] = lv2

                fi32 = fire.astype(jnp.int32)
                cnts[f] = cnt2 - 16 * fi32
                ptrs[f] = ptr + 16 * fi32
            return rem, tuple(cnts) + tuple(ptrs)

        def outer(p, carry):
            p_a = pl.multiple_of(p, 16)
            pltpu.sync_copy(srcb_hbm.at[pl.ds(p_a, SUBB)], sbuf)
            pltpu.sync_copy(dstb_hbm.at[pl.ds(p_a, SUBB)], dbuf)
            rem, cnts_ptrs = carry
            _, cnts_ptrs = lax.fori_loop(
                0, SUBB // 16, vbody, (cen - p, cnts_ptrs))
            return rem, cnts_ptrs

        carry0 = (jnp.int32(0),
                  tuple([jnp.int32(0)] * FPT) + tuple(starts))
        carry = pl.loop(cst, cen, step=SUBB, init_carry=carry0)(outer)
        if carry is None:
            carry = carry0
        _, cnts_ptrs = carry
        cnts = list(cnts_ptrs[:FPT])
        ptrs = list(cnts_ptrs[FPT:])

        for f in range(FPT):
            cnt = cnts[f]
            ptr = ptrs[f]
            sv = sst[pl.ds(f * 48, 16)]
            dv = dst_st[pl.ds(f * 48, 16)]
            sv = jnp.where(lane < cnt, sv, 0)
            dv = jnp.where(lane < cnt, dv, clo + (f + 1) * BT)
            sst[pl.ds(f * 48, 16)] = sv
            dst_st[pl.ds(f * 48, 16)] = dv
            ptr_a = pl.multiple_of(ptr, 16)

            @pl.when(cnt > 0)
            def _():
                pltpu.sync_copy(sst.at[pl.ds(f * 48, 16)],
                                src2_hbm.at[pl.ds(ptr_a, 16)])
                pltpu.sync_copy(dst_st.at[pl.ds(f * 48, 16)],
                                dst2_hbm.at[pl.ds(ptr_a, 16)])

    return k(srcb, dstb, offs1, foff)


def _sc_edge_pass(g, d, src2, dst2, offs2, m16, outinit):
    @functools.partial(
        pl.kernel,
        mesh=_get_mesh(),
        out_type=jax.ShapeDtypeStruct((NPAD, ROWW), jnp.float32),
        scratch_types=[
            pltpu.VMEM((CHK,), jnp.int32),
            pltpu.VMEM((CHK,), jnp.int32),
            pltpu.VMEM((CHK, ROWW), jnp.float32),
            pltpu.VMEM((BT + 16, 4), jnp.float32),
            pltpu.VMEM((16,), jnp.int32),
            pltpu.VMEM((16,), jnp.float32),
            pltpu.VMEM((CHK * 8,), jnp.float32),
            pltpu.VMEM((BT + 16, ROWW), jnp.float32),
            pltpu.SemaphoreType.DMA,
        ],
        name="gat_sc_edge",
        **_SC_PARAMS,
    )
    def k(g_hbm, d_hbm, src2_hbm, dst2_hbm, offs2_hbm, m_hbm, oi_hbm, out_hbm,
          sidx, didx, rows, dblk, o2v, mv, wbuf, acc, sem):
        t = lax.axis_index("c") * 16 + lax.axis_index("s")
        pltpu.sync_copy(offs2_hbm.at[pl.ds(pl.multiple_of(t * 16, 16), 16)], o2v)
        pltpu.sync_copy(m_hbm, mv)
        ovec = o2v[...]
        mvec = mv[...]
        m0 = _extract_f32(mvec, 0)
        m1 = _extract_f32(mvec, 1)
        m2 = _extract_f32(mvec, 2)
        lane = _lane()
        wsel0 = lane == 4
        wsel1 = lane == 5
        wsel2 = lane == 6

        for f in range(FPT):
            lo = pl.multiple_of(t * CO + f * BT, BT)
            pltpu.sync_copy(oi_hbm.at[pl.ds(lo, BT)], acc.at[pl.ds(0, BT)])
            pltpu.sync_copy(d_hbm.at[pl.ds(lo, BT + 16)], dblk)
            st = _extract_i32(ovec, f)
            en = _extract_i32(ovec, f + 1)

            @pl.loop(st, en, step=CHK)
            def _(p):
                p_a = pl.multiple_of(p, 16)
                pltpu.sync_copy(src2_hbm.at[pl.ds(p_a, CHK)], sidx)
                pltpu.sync_copy(dst2_hbm.at[pl.ds(p_a, CHK)], didx)
                rem = en - p
                for j in range(CHK // 16):
                    dv = didx[pl.ds(j * 16, 16)]
                    sv = sidx[pl.ds(j * 16, 16)]
                    valid = (lane + j * 16) < rem
                    dv = jnp.where(valid, dv - lo, jnp.int32(BT))
                    sv = jnp.where(valid, sv, 0)
                    didx[pl.ds(j * 16, 16)] = dv
                    sidx[pl.ds(j * 16, 16)] = sv
                pltpu.async_copy(g_hbm.at[sidx], rows, sem).wait()
                for j in range(CHK // 16):
                    dv = didx[pl.ds(j * 16, 16)]
                    erow = lane + (j * 16)
                    c192 = jnp.full((16,), 192, jnp.int32)
                    asrc0 = plsc.load_gather(rows, [erow, c192])
                    asrc1 = plsc.load_gather(rows, [erow, c192 + 1])
                    asrc2 = plsc.load_gather(rows, [erow, c192 + 2])
                    czero = jnp.zeros((16,), jnp.int32)
                    adst0 = plsc.load_gather(dblk, [dv, czero])
                    adst1 = plsc.load_gather(dblk, [dv, czero + 1])
                    adst2 = plsc.load_gather(dblk, [dv, czero + 2])

                    def wcalc(a, ad, m):
                        e = a + ad
                        e = jnp.where(e > 0, e, 0.2 * e)
                        return jnp.exp(e - m)

                    w0 = wcalc(asrc0, adst0, m0)
                    w1 = wcalc(asrc1, adst1, m1)
                    w2 = wcalc(asrc2, adst2, m2)
                    e8 = erow * 8
                    plsc.store_scatter(wbuf, [e8], w0)
                    plsc.store_scatter(wbuf, [e8 + 1], w1)
                    plsc.store_scatter(wbuf, [e8 + 2], w2)

                @pl.loop(0, CHK, unroll=4)
                def _(e):
                    w0s = plsc.load_gather(wbuf, [jnp.full((16,), e * 8, jnp.int32)])
                    w1s = plsc.load_gather(wbuf, [jnp.full((16,), e * 8 + 1, jnp.int32)])
                    w2s = plsc.load_gather(wbuf, [jnp.full((16,), e * 8 + 2, jnp.int32)])
                    dls = plsc.load_gather(didx, [jnp.full((16,), e, jnp.int32)])
                    ef = jnp.full((16,), e, jnp.int32)
                    ws = (w0s, w1s, w2s)
                    for kk in range(12):
                        col = kk * 16 + lane
                        x = plsc.load_gather(rows, [ef, col])
                        plsc.addupdate_scatter(acc, [dls, col], x * ws[kk // 4])
                    col = 192 + lane
                    x = plsc.load_gather(rows, [ef, col])
                    wsp = jnp.where(wsel0, w0s, jnp.float32(0))
                    wsp = jnp.where(wsel1, w1s, wsp)
                    wsp = jnp.where(wsel2, w2s, wsp)
                    plsc.addupdate_scatter(acc, [dls, col], x * wsp)

            pltpu.sync_copy(acc.at[pl.ds(0, BT)], out_hbm.at[pl.ds(lo, BT)])

    return k(g, d, src2, dst2, offs2, m16, outinit)


def _tc_max1(xp, wa):
    def body(x_ref, wa_ref, m_ref):
        a = jnp.dot(x_ref[...], wa_ref[...], preferred_element_type=jnp.float32)
        bm = jnp.max(a, axis=0, keepdims=True)

        @pl.when(pl.program_id(0) == 0)
        def _():
            m_ref[...] = jnp.full((8, 128), -jnp.inf, jnp.float32)

        m_ref[...] = jnp.maximum(m_ref[...], bm)

    return pl.pallas_call(
        body,
        grid=(_GRID,),
        in_specs=[
            pl.BlockSpec((512, 128), lambda i: (i, 0)),
            pl.BlockSpec((128, 128), lambda i: (0, 0)),
        ],
        out_specs=pl.BlockSpec((8, 128), lambda i: (0, 0)),
        out_shape=jax.ShapeDtypeStruct((8, 128), jnp.float32),
    )(xp, wa)


def _finalize_block(o, brow):
    num = o[:, 0:HC]
    den = o[:, 196:199]
    den3 = lax.broadcast_in_dim(den, (o.shape[0], HEADS, C), (0, 1))
    y = num.reshape(o.shape[0], HEADS, C) / (den3 + 1e-20)
    return y.reshape(o.shape[0], HC) + brow[0:1, :]


def _tc_max2(out1, b1row, wa2):
    def body(o_ref, b_ref, wa_ref, m_ref):
        y = _finalize_block(o_ref[...], b_ref[...])
        a = jnp.dot(y, wa_ref[...], preferred_element_type=jnp.float32)
        bm = jnp.max(a, axis=0, keepdims=True)

        @pl.when(pl.program_id(0) == 0)
        def _():
            m_ref[...] = jnp.full((8, 128), -jnp.inf, jnp.float32)

        m_ref[...] = jnp.maximum(m_ref[...], bm)

    return pl.pallas_call(
        body,
        grid=(_GRID,),
        in_specs=[
            pl.BlockSpec((512, ROWW), lambda i: (i, 0)),
            pl.BlockSpec((8, HC), lambda i: (0, 0)),
            pl.BlockSpec((HC, 128), lambda i: (0, 0)),
        ],
        out_specs=pl.BlockSpec((8, 128), lambda i: (0, 0)),
        out_shape=jax.ShapeDtypeStruct((8, 128), jnp.float32),
    )(out1, b1row, wa2)


def _assemble(h, asrc, adst, w, g_ref, d_ref, oi_ref):
    rows = h.shape[0]
    z1 = jnp.zeros((rows, 1), jnp.float32)
    ztail = jnp.zeros((rows, ROWW - HC - 7), jnp.float32)
    z4 = jnp.zeros((rows, 4), jnp.float32)
    on3 = jnp.ones((rows, 3), jnp.float32)
    g_ref[:, 0:HC] = h
    g_ref[:, HC:ROWW] = jnp.concatenate([asrc, z1, on3, ztail], axis=1)
    d_ref[...] = jnp.concatenate([adst, z1], axis=1)
    w3 = lax.broadcast_in_dim(w, (rows, HEADS, C), (0, 1))
    hw = (h.reshape(rows, HEADS, C) * w3).reshape(rows, HC)
    oi_ref[:, 0:HC] = hw
    oi_ref[:, HC:ROWW] = jnp.concatenate([z4, w, ztail], axis=1)


def _selfw(a, m_ref):
    asrc = a[:, 0:3]
    adst = a[:, 3:6]
    e = asrc + adst
    e = jnp.where(e > 0, e, 0.2 * e)
    m3 = m_ref[0:1, 0:3] + m_ref[0:1, 3:6]
    w = jnp.exp(e - m3)
    return asrc, adst, w


def _tc_prep1(xp, w1p, acat, m8):
    def body(x_ref, w_ref, a_ref, m_ref, g_ref, d_ref, oi_ref):
        x = x_ref[...]
        h = jnp.dot(x, w_ref[...], preferred_element_type=jnp.float32)
        a = jnp.dot(h, a_ref[...], preferred_element_type=jnp.float32)
        asrc, adst, w = _selfw(a, m_ref)
        _assemble(h, asrc, adst, w, g_ref, d_ref, oi_ref)

    return pl.pallas_call(
        body,
        grid=(_GRID,),
        in_specs=[
            pl.BlockSpec((512, 128), lambda i: (i, 0)),
            pl.BlockSpec((128, HC), lambda i: (0, 0)),
            pl.BlockSpec((HC, 128), lambda i: (0, 0)),
            pl.BlockSpec((8, 128), lambda i: (0, 0)),
        ],
        out_specs=[
            pl.BlockSpec((512, ROWW), lambda i: (i, 0)),
            pl.BlockSpec((512, 4), lambda i: (i, 0)),
            pl.BlockSpec((512, ROWW), lambda i: (i, 0)),
        ],
        out_shape=[
            jax.ShapeDtypeStruct((NPAD, ROWW), jnp.float32),
            jax.ShapeDtypeStruct((NPAD, 4), jnp.float32),
            jax.ShapeDtypeStruct((NPAD, ROWW), jnp.float32),
        ],
    )(xp, w1p, acat, m8)


def _tc_prep2(out1, b1row, w2, acat, m8):
    def body(o_ref, b_ref, w_ref, a_ref, m_ref, g_ref, d_ref, oi_ref):
        y = _finalize_block(o_ref[...], b_ref[...])
        h = jnp.dot(y, w_ref[...], preferred_element_type=jnp.float32)
        a = jnp.dot(h, a_ref[...], preferred_element_type=jnp.float32)
        asrc, adst, w = _selfw(a, m_ref)
        _assemble(h, asrc, adst, w, g_ref, d_ref, oi_ref)

    return pl.pallas_call(
        body,
        grid=(_GRID,),
        in_specs=[
            pl.BlockSpec((512, ROWW), lambda i: (i, 0)),
            pl.BlockSpec((8, HC), lambda i: (0, 0)),
            pl.BlockSpec((HC, HC), lambda i: (0, 0)),
            pl.BlockSpec((HC, 128), lambda i: (0, 0)),
            pl.BlockSpec((8, 128), lambda i: (0, 0)),
        ],
        out_specs=[
            pl.BlockSpec((512, ROWW), lambda i: (i, 0)),
            pl.BlockSpec((512, 4), lambda i: (i, 0)),
            pl.BlockSpec((512, ROWW), lambda i: (i, 0)),
        ],
        out_shape=[
            jax.ShapeDtypeStruct((NPAD, ROWW), jnp.float32),
            jax.ShapeDtypeStruct((NPAD, 4), jnp.float32),
            jax.ShapeDtypeStruct((NPAD, ROWW), jnp.float32),
        ],
    )(out1, b1row, w2, acat, m8)


def _tc_head(out2, b2row, wl8, bl8):
    def body(o_ref, b_ref, wl_ref, bl_ref, y_ref):
        y = _finalize_block(o_ref[...], b_ref[...])
        y_ref[...] = (
            jnp.dot(y, wl_ref[...], preferred_element_type=jnp.float32)
            + bl_ref[0:1, :]
        )

    return pl.pallas_call(
        body,
        grid=(_GRID,),
        in_specs=[
            pl.BlockSpec((512, ROWW), lambda i: (i, 0)),
            pl.BlockSpec((8, HC), lambda i: (0, 0)),
            pl.BlockSpec((HC, 8), lambda i: (0, 0)),
            pl.BlockSpec((8, 8), lambda i: (0, 0)),
        ],
        out_specs=pl.BlockSpec((512, 8), lambda i: (i, 0)),
        out_shape=jax.ShapeDtypeStruct((NPAD, 8), jnp.float32),
    )(out2, b2row, wl8, bl8)


def _acat(a_s, a_d):
    blocks = []
    for h in range(HEADS):
        col_s = jnp.zeros((HEADS, C, 1), jnp.float32).at[h, :, 0].set(a_s[0, h])
        blocks.append(col_s.reshape(HC, 1))
    for h in range(HEADS):
        col_d = jnp.zeros((HEADS, C, 1), jnp.float32).at[h, :, 0].set(a_d[0, h])
        blocks.append(col_d.reshape(HC, 1))
    cat = jnp.concatenate(blocks, axis=1)
    return jnp.pad(cat, ((0, 0), (0, 122)))


def _m16(m8):
    m3 = m8[0, 0:3] + m8[0, 3:6]
    return jnp.pad(m3, (0, 13))


def kernel(x, edge_index, W1, a1s, a1d, b1, W2, a2s, a2d, b2, Wl, bl):
    xp = jnp.pad(x, ((0, NPAD - N), (0, 128 - x.shape[1])))
    w1p = jnp.pad(W1, ((0, 128 - W1.shape[0]), (0, 0)))
    acat1 = _acat(a1s, a1d)
    acat2 = _acat(a2s, a2d)
    wa1 = w1p @ acat1
    wa2 = W2 @ acat2
    b1row = jnp.broadcast_to(b1[None, :], (8, HC))
    b2row = jnp.broadcast_to(b2[None, :], (8, HC))
    wl8 = jnp.pad(Wl, ((0, 0), (0, 7)))
    bl8 = jnp.broadcast_to(jnp.pad(bl, (0, 7))[None, :], (8, 8))

    src1d = edge_index[0]
    dst1d = edge_index[1]
    counts = _sc_hist(dst1d).reshape(NW, 32)
    cb = counts.T.reshape(-1).astype(jnp.int32)
    r16 = ((cb + 15) // 16) * 16
    offs1 = jnp.concatenate([jnp.zeros((1,), jnp.int32), jnp.cumsum(r16)])
    offs1 = jnp.pad(offs1, (0, OFFSL1 - NW * NW - 1)).astype(jnp.int32)
    srcb, dstb = _sc_coarse(src1d, dst1d, offs1)
    foff = offs1[jnp.arange(NW) * NW] + jnp.arange(NW, dtype=jnp.int32) * 272
    foff = jnp.pad(foff, (0, 16)).astype(jnp.int32)
    src2, dst2, offs2 = _sc_fine(srcb, dstb, offs1, foff)

    m8_1 = _tc_max1(xp, wa1)
    g1, d1, oi1 = _tc_prep1(xp, w1p, acat1, m8_1)
    d1p = jnp.pad(d1, ((0, 16), (0, 0)))
    out1 = _sc_edge_pass(g1, d1p, src2, dst2, offs2, _m16(m8_1), oi1)

    m8_2 = _tc_max2(out1, b1row, wa2)
    g2, d2, oi2 = _tc_prep2(out1, b1row, W2, acat2, m8_2)
    d2p = jnp.pad(d2, ((0, 16), (0, 0)))
    out2 = _sc_edge_pass(g2, d2p, src2, dst2, offs2, _m16(m8_2), oi2)

    y = _tc_head(out2, b2row, wl8, bl8)
    return y[:N, 0:1]
